# Initial kernel scaffold; baseline (speedup 1.0000x reference)
import jax
import jax.numpy as jnp
from jax.experimental import pallas as pl


def kernel(x, rois):
    raise NotImplementedError("write your pallas kernel here")


# speedup vs baseline: 27.6691x; 27.6691x over previous
import functools

import jax
import jax.numpy as jnp
from jax import lax
from jax.experimental import pallas as pl
from jax.experimental.pallas import tpu as pltpu
from jax.experimental.pallas import tpu_sc as plsc

OUT = 7
SR = 2
SCALE = 0.25
S2 = OUT * OUT * SR * SR * 4
NBIN = OUT * OUT
KPB = SR * SR * 4
NCHUNK = 7
CHUNK = S2 // NCHUNK


def _index_weight_body(rois_ref, idx_ref, w_ref, *, H, W, C):
    bm = rois_ref.shape[0]
    shape = (bm, S2)
    l = lax.broadcasted_iota(jnp.int32, shape, 1)
    bin_ = l // KPB
    k = l % KPB
    sub = k // 4
    corner = k % 4
    by = bin_ // OUT
    bx = bin_ % OUT
    sy = sub // SR
    sx = sub % SR
    cy = corner // 2
    cx = corner % 2
    iy = (by * SR + sy).astype(jnp.float32)
    ix = (bx * SR + sx).astype(jnp.float32)

    b = rois_ref[:, 0:1].astype(jnp.int32)
    rx0 = rois_ref[:, 1:2] * SCALE
    ry0 = rois_ref[:, 2:3] * SCALE
    rx1 = rois_ref[:, 3:4] * SCALE
    ry1 = rois_ref[:, 4:5] * SCALE
    roi_w = jnp.maximum(rx1 - rx0, 1.0)
    roi_h = jnp.maximum(ry1 - ry0, 1.0)
    bin_w = roi_w * (1.0 / OUT)
    bin_h = roi_h * (1.0 / OUT)

    ys = ry0 + (iy + 0.5) * (bin_h * (1.0 / SR))
    xs = rx0 + (ix + 0.5) * (bin_w * (1.0 / SR))
    ys = jnp.clip(ys, 0.0, H - 1.0)
    xs = jnp.clip(xs, 0.0, W - 1.0)
    y0f = jnp.floor(ys)
    x0f = jnp.floor(xs)
    y0 = y0f.astype(jnp.int32)
    x0 = x0f.astype(jnp.int32)
    ly = ys - y0f
    lx = xs - x0f
    y1 = jnp.minimum(y0 + 1, H - 1)
    x1 = jnp.minimum(x0 + 1, W - 1)
    ysel = jnp.where(cy == 1, y1, y0)
    xsel = jnp.where(cx == 1, x1, x0)
    wy = jnp.where(cy == 1, ly, 1.0 - ly)
    wx = jnp.where(cx == 1, lx, 1.0 - lx)
    idx_ref[...] = b * (H * W) + ysel * W + xsel
    w_ref[...] = wy * wx * (1.0 / (SR * SR))


def _make_index_kernel(M, H, W, C):
    bm = 200 if M % 200 == 0 else M
    grid = M // bm
    return pl.pallas_call(
        functools.partial(_index_weight_body, H=H, W=W, C=C),
        grid=(grid,),
        in_specs=[pl.BlockSpec((bm, 5), lambda i: (i, 0))],
        out_specs=[
            pl.BlockSpec((bm, S2), lambda i: (i, 0)),
            pl.BlockSpec((bm, S2), lambda i: (i, 0)),
        ],
        out_shape=[
            jax.ShapeDtypeStruct((M, S2), jnp.int32),
            jax.ShapeDtypeStruct((M, S2), jnp.float32),
        ],
    )


def _make_sc_gather(M, V, C):
    info = plsc.get_sparse_core_info()
    NC, NS = info.num_cores, info.num_subcores
    NW = NC * NS
    rois_per_w = (M + NW - 1) // NW
    mesh = plsc.VectorSubcoreMesh(core_axis_name="c", subcore_axis_name="s")

    @functools.partial(
        pl.kernel,
        mesh=mesh,
        out_type=jax.ShapeDtypeStruct((M, NBIN, C), jnp.float32),
        scratch_types=[
            pltpu.VMEM((NCHUNK, CHUNK), jnp.int32),
            pltpu.VMEM((S2,), jnp.float32),
            pltpu.VMEM((S2, C), jnp.float32),
            pltpu.VMEM((NBIN, C), jnp.float32),
            pltpu.SemaphoreType.DMA,
        ],
    )
    def sc_kernel(xt_hbm, idx_hbm, w_hbm, out_hbm, idx_v, w_v, rows_v, out_v, sem):
        wid = lax.axis_index("s") * NC + lax.axis_index("c")

        def per_roi(j, carry):
            m = j * NW + wid

            @pl.when(m < M)
            def _():
                pltpu.sync_copy(idx_hbm.at[m], idx_v)
                pltpu.sync_copy(w_hbm.at[m], w_v)
                copies = [
                    pltpu.make_async_copy(
                        xt_hbm.at[idx_v.at[c]],
                        rows_v.at[pl.ds(c * CHUNK, CHUNK)],
                        sem,
                    )
                    for c in range(NCHUNK)
                ]
                for cp in copies:
                    cp.start()
                for cp in copies:
                    cp.wait()

                def per_bin(bi, carry2):
                    base = bi * KPB
                    wvec = w_v[pl.ds(base, KPB)]
                    accs = [jnp.zeros((16,), jnp.float32) for _ in range(C // 16)]
                    for k in range(KPB):
                        ws = wvec[k]
                        for r in range(C // 16):
                            accs[r] = accs[r] + ws * rows_v[base + k, pl.ds(r * 16, 16)]
                    for r in range(C // 16):
                        out_v[bi, pl.ds(r * 16, 16)] = accs[r]
                    return carry2

                lax.fori_loop(0, NBIN, per_bin, 0)
                pltpu.sync_copy(out_v, out_hbm.at[m])

            return carry

        lax.fori_loop(0, rois_per_w, per_roi, 0)

    return sc_kernel


def kernel(x, rois):
    B, C, H, W = x.shape
    M = rois.shape[0]
    xt = jnp.transpose(x, (0, 2, 3, 1)).reshape(B * H * W, C)
    idx, w = _make_index_kernel(M, H, W, C)(rois)
    idx3 = idx.reshape(M, NCHUNK, CHUNK)
    out = _make_sc_gather(M, B * H * W, C)(xt, idx3, w)
    return jnp.transpose(out.reshape(M, OUT, OUT, C), (0, 3, 1, 2))


# speedup vs baseline: 35.9455x; 1.2991x over previous
import functools

import jax
import jax.numpy as jnp
from jax import lax
from jax.experimental import pallas as pl
from jax.experimental.pallas import tpu as pltpu
from jax.experimental.pallas import tpu_sc as plsc

OUT = 7
SR = 2
SCALE = 0.25
S2 = OUT * OUT * SR * SR * 4
NBIN = OUT * OUT
KPB = SR * SR * 4
NCHUNK = 7
CHUNK = S2 // NCHUNK


def _index_weight_body(rois_ref, idx_ref, w_ref, *, H, W, C):
    bm = rois_ref.shape[0]
    shape = (bm, S2)
    l = lax.broadcasted_iota(jnp.int32, shape, 1)
    bin_ = l // KPB
    k = l % KPB
    sub = k // 4
    corner = k % 4
    by = bin_ // OUT
    bx = bin_ % OUT
    sy = sub // SR
    sx = sub % SR
    cy = corner // 2
    cx = corner % 2
    iy = (by * SR + sy).astype(jnp.float32)
    ix = (bx * SR + sx).astype(jnp.float32)

    b = rois_ref[:, 0:1].astype(jnp.int32)
    rx0 = rois_ref[:, 1:2] * SCALE
    ry0 = rois_ref[:, 2:3] * SCALE
    rx1 = rois_ref[:, 3:4] * SCALE
    ry1 = rois_ref[:, 4:5] * SCALE
    roi_w = jnp.maximum(rx1 - rx0, 1.0)
    roi_h = jnp.maximum(ry1 - ry0, 1.0)
    bin_w = roi_w * (1.0 / OUT)
    bin_h = roi_h * (1.0 / OUT)

    ys = ry0 + (iy + 0.5) * (bin_h * (1.0 / SR))
    xs = rx0 + (ix + 0.5) * (bin_w * (1.0 / SR))
    ys = jnp.clip(ys, 0.0, H - 1.0)
    xs = jnp.clip(xs, 0.0, W - 1.0)
    y0f = jnp.floor(ys)
    x0f = jnp.floor(xs)
    y0 = y0f.astype(jnp.int32)
    x0 = x0f.astype(jnp.int32)
    ly = ys - y0f
    lx = xs - x0f
    y1 = jnp.minimum(y0 + 1, H - 1)
    x1 = jnp.minimum(x0 + 1, W - 1)
    ysel = jnp.where(cy == 1, y1, y0)
    xsel = jnp.where(cx == 1, x1, x0)
    wy = jnp.where(cy == 1, ly, 1.0 - ly)
    wx = jnp.where(cx == 1, lx, 1.0 - lx)
    idx_ref[...] = b * (H * W) + ysel * W + xsel
    w_ref[...] = wy * wx * (1.0 / (SR * SR))


def _make_index_kernel(M, H, W, C):
    bm = 200 if M % 200 == 0 else M
    grid = M // bm
    return pl.pallas_call(
        functools.partial(_index_weight_body, H=H, W=W, C=C),
        grid=(grid,),
        in_specs=[pl.BlockSpec((bm, 5), lambda i: (i, 0))],
        out_specs=[
            pl.BlockSpec((bm, S2), lambda i: (i, 0)),
            pl.BlockSpec((bm, S2), lambda i: (i, 0)),
        ],
        out_shape=[
            jax.ShapeDtypeStruct((M, S2), jnp.int32),
            jax.ShapeDtypeStruct((M, S2), jnp.float32),
        ],
    )


BPC = CHUNK // KPB
NSLOT = 4


def _make_sc_gather(M, V, C):
    info = plsc.get_sparse_core_info()
    NC, NS = info.num_cores, info.num_subcores
    NW = NC * NS
    rois_per_w = (M + NW - 1) // NW
    R8 = C // 16
    mesh = plsc.VectorSubcoreMesh(core_axis_name="c", subcore_axis_name="s")

    @functools.partial(
        pl.kernel,
        mesh=mesh,
        out_type=jax.ShapeDtypeStruct((M, NBIN, C), jnp.float32),
        scratch_types=[
            pltpu.VMEM((2, NCHUNK, CHUNK), jnp.int32),
            pltpu.VMEM((2, S2), jnp.float32),
            pltpu.VMEM((NSLOT, CHUNK, C), jnp.float32),
            pltpu.VMEM((2, NBIN, C), jnp.float32),
            pltpu.SemaphoreType.DMA,
            pltpu.SemaphoreType.DMA,
            pltpu.SemaphoreType.DMA,
            pltpu.SemaphoreType.DMA,
            pltpu.SemaphoreType.DMA,
            pltpu.SemaphoreType.DMA,
        ],
    )
    def sc_kernel(xt_hbm, idx_hbm, w_hbm, out_hbm, idx_v, w_v, rows_v, out_v,
                  isem, osem, *gsems):
        wid = lax.axis_index("s") * NC + lax.axis_index("c")

        def idx_copies(m, p):
            return (
                pltpu.make_async_copy(idx_hbm.at[m], idx_v.at[p], isem),
                pltpu.make_async_copy(w_hbm.at[m], w_v.at[p], isem),
            )

        def gather(p, c, slot):
            return pltpu.make_async_copy(
                xt_hbm.at[idx_v.at[p, c]], rows_v.at[slot], gsems[slot]
            )

        def compute_chunk(p, c, slot):
            def per_bin(b_, carry2):
                gbin = c * BPC + b_
                wvec = w_v[p, pl.ds(gbin * KPB, KPB)]
                base = b_ * KPB
                accs = [jnp.zeros((16,), jnp.float32) for _ in range(R8)]
                for k in range(KPB):
                    ws = wvec[k]
                    for r in range(R8):
                        accs[r] = accs[r] + ws * rows_v[slot, base + k, pl.ds(r * 16, 16)]
                for r in range(R8):
                    out_v[p, gbin, pl.ds(r * 16, 16)] = accs[r]
                return carry2

            lax.fori_loop(0, BPC, per_bin, 0)

        def per_roi(j, carry):
            m = j * NW + wid
            p = j % 2

            @pl.when(m < M)
            def _():
                @pl.when(j > 0)
                def _():
                    a, b = idx_copies(m, p)
                    a.wait()
                    b.wait()

                @pl.when(m + NW < M)
                def _():
                    a, b = idx_copies(m + NW, 1 - p)
                    a.start()
                    b.start()

                for c in range(NSLOT):
                    gather(p, c, c).start()

                @pl.when(j >= 2)
                def _():
                    pltpu.make_async_copy(out_v.at[p], out_hbm.at[m], osem).wait()

                for c in range(NCHUNK):
                    slot = c % NSLOT
                    gather(p, c, slot).wait()
                    compute_chunk(p, c, slot)
                    if c + NSLOT < NCHUNK:
                        gather(p, c + NSLOT, slot).start()

                pltpu.make_async_copy(out_v.at[p], out_hbm.at[m], osem).start()

            return carry

        pltpu.sync_copy(idx_hbm.at[wid], idx_v.at[0])
        pltpu.sync_copy(w_hbm.at[wid], w_v.at[0])

        lax.fori_loop(0, rois_per_w, per_roi, 0)

        nj = (M - 1 - wid) // NW + 1
        for t in range(2):
            @pl.when(nj > t)
            def _():
                pltpu.make_async_copy(out_v.at[0], out_hbm.at[0], osem).wait()

    return sc_kernel


def kernel(x, rois):
    B, C, H, W = x.shape
    M = rois.shape[0]
    xt = jnp.transpose(x, (0, 2, 3, 1)).reshape(B * H * W, C)
    idx, w = _make_index_kernel(M, H, W, C)(rois)
    idx3 = idx.reshape(M, NCHUNK, CHUNK)
    out = _make_sc_gather(M, B * H * W, C)(xt, idx3, w)
    return jnp.transpose(out.reshape(M, OUT, OUT, C), (0, 3, 1, 2))
